# baseline (speedup 1.0000x reference)
import functools

import jax
import jax.numpy as jnp
from jax import lax
from jax.experimental import pallas as pl
from jax.experimental.pallas import tpu as pltpu
from jax.experimental.pallas import tpu_sc as plsc

VOCAB = 1000000
EMB_DIM = 64
BATCH = 4096
HIST = 50

_NC = 2
_NS = 16
_NW = _NC * _NS

_W = 384
_NFULL = (VOCAB - 64) // _W
_APT = -(-_NFULL // _NW)
_TAIL0 = VOCAB - 64


def _transpose_kernel(tt_hbm, tail_hbm, t2_hbm,
                      in0, in1, ob0, ob1, tail_v, gs0, gs1, ws0, ws1):
    wid = lax.axis_index("s") * _NC + lax.axis_index("c")
    ins = (in0, in1)
    obs = (ob0, ob1)
    gsems = (gs0, gs1)
    wsems = (ws0, ws1)
    iota64 = lax.iota(jnp.int32, 16) * EMB_DIM

    def fire(i, s):
        k = i * _NW + wid

        @pl.when(k < _NFULL)
        def _():
            for ehi in range(8):
                pltpu.async_copy(
                    tt_hbm.at[pl.ds(8 * ehi, 8), pl.ds(k * _W, _W)],
                    ins[s].at[pl.ds(8 * ehi, 8)], gsems[s])

    fire(0, 0)

    def block(i, _):
        def run(s):
            k = i * _NW + wid

            @pl.when(k < _NFULL)
            def _():
                for ehi in range(8):
                    pltpu.make_async_copy(
                        tt_hbm.at[pl.ds(8 * ehi, 8), pl.ds(k * _W, _W)],
                        ins[s].at[pl.ds(8 * ehi, 8)], gsems[s]).wait()
                fire(i + 1, 1 - s)

                @pl.when(i >= 2)
                def _():
                    pltpu.make_async_copy(
                        obs[s],
                        t2_hbm.at[pl.ds((k - 2 * _NW) * _W * EMB_DIM,
                                        _W * EMB_DIM)],
                        wsems[s]).wait()

                inb = ins[s]
                ob = obs[s]

                @plsc.parallel_loop(0, EMB_DIM, unroll=4)
                def _(e):
                    for v24 in range(_W // 16):
                        v = inb[e, pl.ds(v24 * 16, 16)]
                        plsc.store_scatter(
                            ob, [iota64 + (v24 * 16 * EMB_DIM + e)], v)
                pltpu.async_copy(
                    ob, t2_hbm.at[pl.ds(k * _W * EMB_DIM, _W * EMB_DIM)],
                    wsems[s])

        lax.cond(lax.rem(i, 2) == 0, lambda: run(0), lambda: run(1))
        return _

    lax.fori_loop(0, _APT, block, 0)

    def drain(s, i):
        k = i * _NW + wid

        @pl.when((k < _NFULL) & (i >= 0))
        def _():
            pltpu.make_async_copy(
                obs[s], t2_hbm.at[pl.ds(k * _W * EMB_DIM, _W * EMB_DIM)],
                wsems[s]).wait()

    drain((_APT - 2) % 2, _APT - 2)
    drain((_APT - 1) % 2, _APT - 1)

    @pl.when(wid == 0)
    def _():
        pltpu.sync_copy(tail_hbm, tail_v)

        def erow_t(e, _):
            for v16 in range(4):
                v = plsc.load_gather(
                    tail_v, [lax.iota(jnp.int32, 16) + v16 * 16,
                             jnp.full((16,), e, jnp.int32)])
                plsc.store_scatter(
                    ob0, [iota64 + (v16 * 16 * EMB_DIM + e)], v)
            return _

        lax.fori_loop(0, EMB_DIM, erow_t, 0)
        pltpu.sync_copy(
            ob0.at[pl.ds(0, 64 * EMB_DIM)],
            t2_hbm.at[pl.ds(_TAIL0 * EMB_DIM, 64 * EMB_DIM)])


_BB = BATCH // _NW
_BPW = HIST * _BB
_HC = 2
_NCHUNK = HIST // _HC
_ROWS = _HC * _BB


def _gather_kernel(xp_hbm, table_hbm, out_hbm, idx_v, idx2_v,
                   rows0, rows1, dst0, dst1, gsem0, gsem1, wsem0, wsem1):
    wid = lax.axis_index("s") * _NC + lax.axis_index("c")
    pltpu.sync_copy(xp_hbm.at[wid], idx_v)

    @plsc.parallel_loop(0, _BPW // 16, unroll=4)
    def _(i):
        idx2_v[pl.ds(i * 16, 16)] = lax.shift_right_logical(
            idx_v[0, pl.ds(i * 16, 16)], 1)

    rows = (rows0, rows1)
    dsts = (dst0, dst1)
    gsems = (gsem0, gsem1)
    wsems = (wsem0, wsem1)

    def fire(c, s):
        pltpu.async_copy(
            table_hbm.at[idx2_v.at[pl.ds(c * _ROWS, _ROWS)]],
            rows[s], gsems[s])

    fire(0, 0)

    iota16 = lax.iota(jnp.int32, 16)

    def chunk(c, _):
        def run(s):
            rows_v = rows[s]
            dst_v = dsts[s]
            pltpu.make_async_copy(
                table_hbm.at[idx2_v.at[pl.ds(c * _ROWS, _ROWS)]],
                rows_v, gsems[s]).wait()

            @pl.when(c + 1 < _NCHUNK)
            def _():
                fire(c + 1, 1 - s)

            @pl.when(c >= 2)
            def _():
                pltpu.make_async_copy(
                    dst_v, out_hbm.at[pl.ds((c - 2) * _HC, _HC), :, wid],
                    wsems[s]).wait()

            @plsc.parallel_loop(0, _HC * 8, unroll=2)
            def _(m):
                h = m // 8
                b16 = m % 8
                r16 = h * _BB + b16 * 16 + iota16
                par = lax.bitwise_and(
                    plsc.load_gather(idx_v, [jnp.zeros((16,), jnp.int32),
                                             c * _ROWS + r16]), 1)
                colbase = par * EMB_DIM
                for e in range(EMB_DIM):
                    v = plsc.load_gather(rows_v, [r16, colbase + e])
                    dst_v[h, e // 8, e % 8, pl.ds(b16 * 16, 16)] = v
            pltpu.async_copy(
                dst_v, out_hbm.at[pl.ds(c * _HC, _HC), :, wid], wsems[s])

        lax.cond(lax.rem(c, 2) == 0, lambda: run(0), lambda: run(1))
        return _

    lax.fori_loop(0, _NCHUNK, chunk, 0)

    def drain(s, c):
        pltpu.make_async_copy(
            dsts[s], out_hbm.at[pl.ds(c * _HC, _HC), :, wid],
            wsems[s]).wait()
    drain(_NCHUNK % 2, _NCHUNK - 2)
    drain((_NCHUNK + 1) % 2, _NCHUNK - 1)


_MESH = plsc.VectorSubcoreMesh(core_axis_name="c", subcore_axis_name="s")


@jax.jit
def _embed(x, table):
    tt = table.T
    tail = table[_TAIL0:, :]
    xp = (x.T.reshape(_NCHUNK, _HC, _NW, _BB)
          .transpose(2, 0, 1, 3).reshape(_NW, 1, _BPW))

    ta = functools.partial(
        pl.kernel,
        mesh=_MESH,
        out_type=jax.ShapeDtypeStruct((VOCAB * EMB_DIM,), jnp.float32),
        scratch_types=[
            pltpu.VMEM((EMB_DIM, _W), jnp.float32),
            pltpu.VMEM((EMB_DIM, _W), jnp.float32),
            pltpu.VMEM((_W * EMB_DIM,), jnp.float32),
            pltpu.VMEM((_W * EMB_DIM,), jnp.float32),
            pltpu.VMEM((64, EMB_DIM), jnp.float32),
            pltpu.SemaphoreType.DMA,
            pltpu.SemaphoreType.DMA,
            pltpu.SemaphoreType.DMA,
            pltpu.SemaphoreType.DMA,
        ],
        compiler_params=pltpu.CompilerParams(
            use_tc_tiling_on_sc=True, needs_layout_passes=False),
    )(_transpose_kernel)
    t2 = ta(tt, tail).reshape(VOCAB // 2, 2 * EMB_DIM)

    tb = functools.partial(
        pl.kernel,
        mesh=_MESH,
        out_type=jax.ShapeDtypeStruct(
            (HIST, EMB_DIM // 8, _NW, 8, _BB), jnp.float32),
        scratch_types=[
            pltpu.VMEM((1, _BPW), jnp.int32),
            pltpu.VMEM((_BPW,), jnp.int32),
            pltpu.VMEM((_ROWS, 2 * EMB_DIM), jnp.float32),
            pltpu.VMEM((_ROWS, 2 * EMB_DIM), jnp.float32),
            pltpu.VMEM((_HC, 8, 8, _BB), jnp.float32),
            pltpu.VMEM((_HC, 8, 8, _BB), jnp.float32),
            pltpu.SemaphoreType.DMA,
            pltpu.SemaphoreType.DMA,
            pltpu.SemaphoreType.DMA,
            pltpu.SemaphoreType.DMA,
        ],
        compiler_params=pltpu.CompilerParams(
            use_tc_tiling_on_sc=True, needs_layout_passes=False),
    )(_gather_kernel)
    out5d = tb(xp, t2)
    return out5d.transpose(2, 4, 0, 1, 3).reshape(BATCH, HIST, EMB_DIM)


def kernel(x, table):
    return _embed(x, table)


# speedup vs baseline: 1.4101x; 1.4101x over previous
import functools

import jax
import jax.numpy as jnp
from jax import lax
from jax.experimental import pallas as pl
from jax.experimental.pallas import tpu as pltpu
from jax.experimental.pallas import tpu_sc as plsc

VOCAB = 1000000
EMB_DIM = 64
BATCH = 4096
HIST = 50

_B = BATCH * HIST
_NC = 2
_NS = 16
_NW = _NC * _NS
_BPW = _B // _NW
_NBUF = 4
_CH = 400
_NCHUNK = _BPW // _CH


def _gather_kernel(x_hbm, table_hbm, out_hbm, idx_v, bufs, sems):
    wid = lax.axis_index("s") * _NC + lax.axis_index("c")
    base = wid * _BPW
    pltpu.sync_copy(x_hbm.at[pl.ds(base, _BPW)], idx_v)

    def fire(c):
        s = c % _NBUF
        pltpu.async_copy(
            table_hbm.at[idx_v.at[pl.ds(c * _CH, _CH)]], bufs[s], sems[s])

    for c in range(_NBUF):
        fire(c)
    for c in range(_NCHUNK):
        s = c % _NBUF
        pltpu.make_async_copy(
            table_hbm.at[idx_v.at[pl.ds(c * _CH, _CH)]], bufs[s], sems[s]
        ).wait()
        pltpu.sync_copy(bufs[s], out_hbm.at[pl.ds(base + c * _CH, _CH)])
        if c + _NBUF < _NCHUNK:
            fire(c + _NBUF)


@jax.jit
def _embed(x_flat, table):
    mesh = plsc.VectorSubcoreMesh(core_axis_name="c", subcore_axis_name="s")
    f = functools.partial(
        pl.kernel,
        mesh=mesh,
        out_type=jax.ShapeDtypeStruct((_B, EMB_DIM), jnp.float32),
        scratch_types=[
            pltpu.VMEM((_BPW,), jnp.int32),
            [pltpu.VMEM((_CH, EMB_DIM), jnp.float32) for _ in range(_NBUF)],
            [pltpu.SemaphoreType.DMA for _ in range(_NBUF)],
        ],
        compiler_params=pltpu.CompilerParams(use_tc_tiling_on_sc=False),
    )(_gather_kernel)
    return f(x_flat, table)


def kernel(x, table):
    out = _embed(x.reshape(_B), table)
    return out.reshape(BATCH, HIST, EMB_DIM)


# speedup vs baseline: 1.4220x; 1.0084x over previous
import functools

import jax
import jax.numpy as jnp
from jax import lax
from jax.experimental import pallas as pl
from jax.experimental.pallas import tpu as pltpu
from jax.experimental.pallas import tpu_sc as plsc

VOCAB = 1000000
EMB_DIM = 64
BATCH = 4096
HIST = 50

_B = BATCH * HIST
_NC = 2
_NS = 16
_NW = _NC * _NS
_BPW = _B // _NW
_NBUF = 4
_CH = 200
_NCHUNK = _BPW // _CH


def _gather_kernel(x_hbm, table_hbm, out_hbm, idx_v, bufs, sems):
    wid = lax.axis_index("s") * _NC + lax.axis_index("c")
    base = wid * _BPW
    pltpu.sync_copy(x_hbm.at[0, pl.ds(base, _BPW)], idx_v)

    def fire(c):
        s = c % _NBUF
        pltpu.async_copy(
            table_hbm.at[idx_v.at[pl.ds(c * _CH, _CH)]], bufs[s], sems[s])

    for c in range(_NBUF):
        fire(c)
    for c in range(_NCHUNK):
        s = c % _NBUF
        pltpu.make_async_copy(
            table_hbm.at[idx_v.at[pl.ds(c * _CH, _CH)]], bufs[s], sems[s]
        ).wait()
        pltpu.sync_copy(bufs[s], out_hbm.at[pl.ds(base + c * _CH, _CH)])
        if c + _NBUF < _NCHUNK:
            fire(c + _NBUF)


@jax.jit
def _embed(x_flat, tpad):
    mesh = plsc.VectorSubcoreMesh(core_axis_name="c", subcore_axis_name="s")
    f = functools.partial(
        pl.kernel,
        mesh=mesh,
        out_type=jax.ShapeDtypeStruct((_B, 2 * EMB_DIM), jnp.float32),
        scratch_types=[
            pltpu.VMEM((_BPW,), jnp.int32),
            [pltpu.VMEM((_CH, 2 * EMB_DIM), jnp.float32)
             for _ in range(_NBUF)],
            [pltpu.SemaphoreType.DMA for _ in range(_NBUF)],
        ],
        compiler_params=pltpu.CompilerParams(
            use_tc_tiling_on_sc=True, needs_layout_passes=False),
    )(_gather_kernel)
    return f(x_flat, tpad)


def kernel(x, table):
    tpad = jnp.pad(table, ((0, 0), (0, EMB_DIM)))
    out = _embed(x.reshape(1, _B), tpad)
    return out[:, :EMB_DIM].reshape(BATCH, HIST, EMB_DIM)
